# Initial kernel scaffold; baseline (speedup 1.0000x reference)
import jax
import jax.numpy as jnp
from jax.experimental import pallas as pl


def kernel(x, edge_index, W1l, b1, W1r, g1, be1, W2l, b2, W2r, g2, be2, W3l, b3, W3r):
    raise NotImplementedError("write your pallas kernel here")


# speedup vs baseline: 4.2444x; 4.2444x over previous
import jax
import jax.numpy as jnp
from jax import lax
from jax.experimental import pallas as pl
from jax.experimental.pallas import tpu as pltpu
from jax.experimental.pallas import tpu_sc as plsc

N = 10000
D = 128
NC = 2
NS = 16
NW = NC * NS
CHUNK = 128
N_PAD = 10240
ROWS_PER_TILE = N_PAD // NS
CNT_W = 8

_MESH = plsc.VectorSubcoreMesh(
    core_axis_name="c", subcore_axis_name="s",
    num_cores=NC, num_subcores=NS)


def _make_sc_agg(nch: int):
    scratch = [
        pltpu.VMEM((nch, CHUNK), jnp.int32),
        pltpu.VMEM((nch, CHUNK), jnp.int32),
        pltpu.VMEM((CHUNK, D), jnp.float32),
        pltpu.VMEM_SHARED((N_PAD, D), jnp.float32),
        pltpu.SemaphoreType.DMA,
    ]

    def body(h_hbm, src_hbm, dst_hbm, zrow_hbm, agg_out,
             src_v, dst_v, rows_v, agg_sh, sem):
        c = lax.axis_index("c")
        s = lax.axis_index("s")
        base = s * ROWS_PER_TILE

        pltpu.sync_copy(src_hbm.at[c, s], src_v)
        pltpu.sync_copy(dst_hbm.at[c, s], dst_v)

        pltpu.sync_copy(zrow_hbm, rows_v)
        for k in range(ROWS_PER_TILE // CHUNK):
            pltpu.sync_copy(rows_v, agg_sh.at[pl.ds(base + k * CHUNK, CHUNK)])
        plsc.subcore_barrier()

        def step(j, carry):
            pltpu.async_copy(h_hbm.at[src_v.at[j]], rows_v, sem).wait()
            pltpu.sync_copy(rows_v, agg_sh.at[dst_v.at[j]], add=True)
            return carry

        lax.fori_loop(0, nch, step, 0)
        plsc.subcore_barrier()

        pltpu.sync_copy(agg_sh.at[pl.ds(base, ROWS_PER_TILE)],
                        agg_out.at[c, pl.ds(base, ROWS_PER_TILE)])

    return pl.kernel(
        body,
        out_type=jax.ShapeDtypeStruct((NC, N_PAD, D), jnp.float32),
        mesh=_MESH, scratch_types=scratch)


def _make_sc_cnt(nch: int):
    scratch = [
        pltpu.VMEM((nch, CHUNK), jnp.int32),
        pltpu.VMEM((CHUNK, D), jnp.float32),
        pltpu.VMEM_SHARED((N_PAD, D), jnp.float32),
    ]

    def body(dst_hbm, zrow_hbm, ones_hbm, cnt_out, dst_v, ones_v, cnt_sh):
        c = lax.axis_index("c")
        s = lax.axis_index("s")
        base = s * ROWS_PER_TILE

        pltpu.sync_copy(dst_hbm.at[c, s], dst_v)
        pltpu.sync_copy(zrow_hbm, ones_v)
        for k in range(ROWS_PER_TILE // CHUNK):
            pltpu.sync_copy(ones_v, cnt_sh.at[pl.ds(base + k * CHUNK, CHUNK)])
        pltpu.sync_copy(ones_hbm, ones_v)
        plsc.subcore_barrier()

        def step(j, carry):
            pltpu.sync_copy(ones_v, cnt_sh.at[dst_v.at[j]], add=True)
            return carry

        lax.fori_loop(0, nch, step, 0)
        plsc.subcore_barrier()

        pltpu.sync_copy(cnt_sh.at[pl.ds(base, ROWS_PER_TILE)],
                        cnt_out.at[c, pl.ds(base, ROWS_PER_TILE)])

    return pl.kernel(
        body,
        out_type=jax.ShapeDtypeStruct((NC, N_PAD, D), jnp.float32),
        mesh=_MESH, scratch_types=scratch)


def _dense_layer(parts, cnt_parts, h, Wl, b, Wr, g, be, with_bn: bool):
    def body(parts_ref, cnt_ref, h_ref, wl_ref, wr_ref, b_ref, g_ref,
             be_ref, o_ref):
        cnt = cnt_ref[0, :N, 0:1] + cnt_ref[1, :N, 0:1]
        inv = 1.0 / jnp.maximum(cnt, 1.0)
        mean = (parts_ref[0, :N, :] + parts_ref[1, :N, :]) * inv
        z = (jnp.dot(mean, wl_ref[...], preferred_element_type=jnp.float32)
             + jnp.dot(h_ref[...], wr_ref[...],
                       preferred_element_type=jnp.float32)
             + b_ref[...])
        if with_bn:
            mu = jnp.mean(z, axis=0, keepdims=True)
            var = jnp.mean((z - mu) ** 2, axis=0, keepdims=True)
            z = g_ref[...] * (z - mu) / jnp.sqrt(var + 1e-5) + be_ref[...]
            z = jnp.maximum(z, 0.0)
        o_ref[...] = z

    if g is None:
        g = jnp.zeros((D,), jnp.float32)
        be = jnp.zeros((D,), jnp.float32)
    return pl.pallas_call(
        body,
        out_shape=jax.ShapeDtypeStruct((N, D), jnp.float32),
    )(parts, cnt_parts, h, Wl, Wr, b.reshape(1, D), g.reshape(1, D),
      be.reshape(1, D))


def kernel(x, edge_index, W1l, b1, W1r, g1, be1, W2l, b2, W2r, g2, be2,
           W3l, b3, W3r):
    src = edge_index[0]
    dst = edge_index[1]
    E = src.shape[0]
    nch = -(-E // (NW * CHUNK))
    e_pad = NW * nch * CHUNK
    src_p = jnp.concatenate(
        [src, jnp.zeros((e_pad - E,), jnp.int32)]).reshape(NC, NS, nch, CHUNK)
    dst_p = jnp.concatenate(
        [dst, jnp.full((e_pad - E,), N, jnp.int32)]).reshape(NC, NS, nch, CHUNK)

    zrow = jnp.zeros((CHUNK, D), jnp.float32)
    ones = jnp.ones((CHUNK, D), jnp.float32)

    agg_fn = _make_sc_agg(nch)
    cnt_fn = _make_sc_cnt(nch)

    cnt_p = cnt_fn(dst_p, zrow, ones)
    agg1 = agg_fn(x, src_p, dst_p, zrow)
    h1 = _dense_layer(agg1, cnt_p, x, W1l, b1, W1r, g1, be1, with_bn=True)
    agg2 = agg_fn(h1, src_p, dst_p, zrow)
    h2 = _dense_layer(agg2, cnt_p, h1, W2l, b2, W2r, g2, be2, with_bn=True)
    agg3 = agg_fn(h2, src_p, dst_p, zrow)
    return _dense_layer(agg3, cnt_p, h2, W3l, b3, W3r, None, None,
                        with_bn=False)
